# baseline (speedup 1.0000x reference)
import functools

import jax
import jax.numpy as jnp
from jax import lax
from jax.experimental import pallas as pl
from jax.experimental.pallas import tpu as pltpu
from jax.experimental.pallas import tpu_sc as plsc

B, S, D, E, K = 4, 2048, 2048, 16, 2
NOISY_STD = 1.0
T = 1024

_info = plsc.get_sparse_core_info()
_NC, _NS = _info.num_cores, _info.num_subcores
_NW = _NC * _NS


def _matmul_body(x_ref, wt_ref, nw_ref, noise_ref, l_out_ref):
    logits = jax.lax.dot_general(
        x_ref[...], wt_ref[...],
        (((1,), (0,)), ((), ())),
        preferred_element_type=jnp.float32,
    )
    l_out_ref[...] = logits + noise_ref[...] * (NOISY_STD * nw_ref[...])


def _sc_gate(logits_hbm, w_out_hbm, idx_out_hbm, lbuf, wbuf, ibuf):
    rows_w = logits_hbm.shape[0] // _NW
    wid = lax.axis_index("s") * _NC + lax.axis_index("c")
    base = wid * rows_w
    pltpu.sync_copy(logits_hbm.at[pl.ds(base, rows_w), :], lbuf)

    iota = lax.iota(jnp.int32, E)
    neg_inf = jnp.float32(-jnp.inf)
    perms = [jnp.bitwise_xor(iota, jnp.int32(1 << p)) for p in range(4)]

    def _shuf(v, perm):
        return v.at[perm].get(mode="promise_in_bounds")

    def _allmax(v):
        for perm in perms:
            v = jnp.maximum(v, _shuf(v, perm))
        return v

    def _allmin(v):
        for perm in perms:
            v = jnp.minimum(v, _shuf(v, perm))
        return v

    def body(r, carry):
        lrow = lbuf[r]
        m1 = _allmax(lrow)
        idx1 = _allmin(jnp.where(lrow == m1, iota, E))
        masked = jnp.where(iota == idx1, neg_inf, lrow)
        m2 = _allmax(masked)
        idx2 = _allmin(jnp.where(masked == m2, iota, E))
        e2 = jnp.exp(m2 - m1)
        w1 = 1.0 / (1.0 + e2)
        w2 = e2 * w1
        wbuf[r] = jnp.where(iota == idx1, w1,
                            jnp.where(iota == idx2, w2, jnp.float32(0.0)))
        ibuf[r] = jnp.where(iota == 0, idx1, jnp.where(iota == 1, idx2, 0))
        return carry

    lax.fori_loop(0, rows_w, body, 0)

    pltpu.sync_copy(wbuf, w_out_hbm.at[pl.ds(base, rows_w), :])
    pltpu.sync_copy(ibuf, idx_out_hbm.at[pl.ds(base, rows_w), :])


@jax.jit
def kernel(x, W, noise_weight, noise):
    n = B * S
    x2 = x.reshape(n, D)
    wt = W.T
    nw = noise_weight.reshape(1, E)
    noise2 = noise.reshape(n, E)

    grid = (n // T,)
    logits = pl.pallas_call(
        _matmul_body,
        grid=grid,
        in_specs=[
            pl.BlockSpec((T, D), lambda i: (i, 0)),
            pl.BlockSpec((D, E), lambda i: (0, 0)),
            pl.BlockSpec((1, E), lambda i: (0, 0)),
            pl.BlockSpec((T, E), lambda i: (i, 0)),
        ],
        out_specs=pl.BlockSpec((T, E), lambda i: (i, 0)),
        out_shape=jax.ShapeDtypeStruct((n, E), jnp.float32),
        compiler_params=pltpu.CompilerParams(
            dimension_semantics=("arbitrary",),
        ),
    )(x2, wt, nw, noise2)

    rows_w = n // _NW
    mesh = plsc.VectorSubcoreMesh(core_axis_name="c", subcore_axis_name="s")
    gate = functools.partial(
        pl.kernel,
        mesh=mesh,
        out_type=[
            jax.ShapeDtypeStruct((n, E), jnp.float32),
            jax.ShapeDtypeStruct((n, E), jnp.int32),
        ],
        scratch_types=[
            pltpu.VMEM((rows_w, E), jnp.float32),
            pltpu.VMEM((rows_w, E), jnp.float32),
            pltpu.VMEM((rows_w, E), jnp.int32),
        ],
    )(_sc_gate)
    weights, idx16 = gate(logits)

    return weights.reshape(B, S, E), idx16[:, :K].reshape(B, S, K)


# speedup vs baseline: 1.5057x; 1.5057x over previous
import jax
import jax.numpy as jnp
from jax.experimental import pallas as pl
from jax.experimental.pallas import tpu as pltpu

B, S, D, E, K = 4, 2048, 2048, 16, 2
NOISY_STD = 1.0
T = 1024


def _gate_body(x_ref, wt_ref, nw_ref, noise_ref, w_out_ref, idx_out_ref):
    logits = jax.lax.dot_general(
        x_ref[...], wt_ref[...],
        (((1,), (0,)), ((), ())),
        preferred_element_type=jnp.float32,
    )
    logits = logits + noise_ref[...] * (NOISY_STD * nw_ref[...])

    iota = jax.lax.broadcasted_iota(jnp.int32, (T, E), 1)
    neg_inf = jnp.float32(-jnp.inf)

    m1 = jnp.max(logits, axis=1, keepdims=True)
    idx1 = jnp.min(jnp.where(logits == m1, iota, E), axis=1, keepdims=True)
    masked = jnp.where(iota == idx1, neg_inf, logits)
    m2 = jnp.max(masked, axis=1, keepdims=True)
    idx2 = jnp.min(jnp.where(masked == m2, iota, E), axis=1, keepdims=True)

    e2 = jnp.exp(m2 - m1)
    denom = 1.0 + e2
    w1 = 1.0 / denom
    w2 = e2 / denom

    w_out_ref[...] = jnp.where(
        iota == idx1, w1, jnp.where(iota == idx2, w2, jnp.float32(0.0)))
    idx_out_ref[...] = jnp.concatenate([idx1, idx2], axis=1)


@jax.jit
def kernel(x, W, noise_weight, noise):
    n = B * S
    x2 = x.reshape(n, D)
    wt = W.T
    nw = noise_weight.reshape(1, E)
    noise2 = noise.reshape(n, E)

    grid = (n // T,)
    weights, idx = pl.pallas_call(
        _gate_body,
        grid=grid,
        in_specs=[
            pl.BlockSpec((T, D), lambda i: (i, 0)),
            pl.BlockSpec((D, E), lambda i: (0, 0)),
            pl.BlockSpec((1, E), lambda i: (0, 0)),
            pl.BlockSpec((T, E), lambda i: (i, 0)),
        ],
        out_specs=[
            pl.BlockSpec((T, E), lambda i: (i, 0)),
            pl.BlockSpec((T, K), lambda i: (i, 0)),
        ],
        out_shape=[
            jax.ShapeDtypeStruct((n, E), jnp.float32),
            jax.ShapeDtypeStruct((n, K), jnp.int32),
        ],
        compiler_params=pltpu.CompilerParams(
            dimension_semantics=("arbitrary",),
        ),
    )(x2, wt, nw, noise2)

    return weights.reshape(B, S, E), idx.reshape(B, S, K)
